# baseline (speedup 1.0000x reference)
import jax
import jax.numpy as jnp
from jax import lax
from jax.experimental import pallas as pl
from jax.experimental.pallas import tpu as pltpu

_TOP_K = 2
_NEG_INF = float("-inf")


def _router_body(xt_ref, wr_ref, br_ref, wn_ref, bn_ref, noise_ref,
                 idx_ref, gate_ref):
    xs = xt_ref[...]
    pooled = jnp.mean(xs, axis=1)
    logits = jnp.dot(pooled, wr_ref[...],
                     preferred_element_type=jnp.float32) + br_ref[0]
    nlog = jnp.dot(pooled, wn_ref[...],
                   preferred_element_type=jnp.float32) + bn_ref[0]
    noisy = logits + noise_ref[...] * jax.nn.softplus(nlog)

    bs, ne = noisy.shape
    eids = lax.broadcasted_iota(jnp.int32, (bs, ne), 1)
    v0 = jnp.max(noisy, axis=1)
    i0 = jnp.min(jnp.where(noisy == v0[:, None], eids, ne), axis=1)
    masked = jnp.where(eids == i0[:, None], _NEG_INF, noisy)
    v1 = jnp.max(masked, axis=1)
    i1 = jnp.min(jnp.where(masked == v1[:, None], eids, ne), axis=1)
    t = jnp.exp(v1 - v0)
    g0 = 1.0 / (1.0 + t)
    g1 = t / (1.0 + t)
    idx_ref[...] = jnp.concatenate([i0[None, :], i1[None, :]], axis=0)
    gate_ref[...] = jnp.concatenate([g0[None, :], g1[None, :]], axis=0)


def _expert_body(idx_ref, gate_ref, xt_ref, w1_ref, b1_ref, w2_ref, b2_ref,
                 out_ref):
    b = pl.program_id(0)
    k = pl.program_id(1)
    e = idx_ref[k, b]
    g = gate_ref[k, b]
    xb = xt_ref[0]
    h1 = jnp.dot(xb, w1_ref[0], preferred_element_type=jnp.float32)
    h1 = jnp.maximum(h1 + b1_ref[pl.ds(e, 1), :], 0.0)
    h2 = jnp.dot(h1, w2_ref[0], preferred_element_type=jnp.float32)
    val = g * (h2 + b2_ref[pl.ds(e, 1), :])

    @pl.when(k == 0)
    def _():
        out_ref[0] = val

    @pl.when(k != 0)
    def _():
        out_ref[0] += val


def kernel(x, Wr, br, Wn, bn, W1, b1, W2, b2):
    bs, dim, h, w = x.shape
    hw = h * w
    ne = Wr.shape[1]
    hid = W1.shape[2]

    xt = jnp.transpose(x, (0, 2, 3, 1)).reshape(bs, hw, dim)
    noise = jax.random.normal(jax.random.key(42), (bs, ne), dtype=jnp.float32)

    idx, gates = pl.pallas_call(
        _router_body,
        out_shape=(
            jax.ShapeDtypeStruct((_TOP_K, bs), jnp.int32),
            jax.ShapeDtypeStruct((_TOP_K, bs), jnp.float32),
        ),
    )(xt, Wr, br.reshape(1, ne), Wn, bn.reshape(1, ne), noise)

    grid_spec = pltpu.PrefetchScalarGridSpec(
        num_scalar_prefetch=2,
        grid=(bs, _TOP_K),
        in_specs=[
            pl.BlockSpec((1, hw, dim), lambda b, k, i_ref, g_ref: (b, 0, 0)),
            pl.BlockSpec((1, dim, hid),
                         lambda b, k, i_ref, g_ref: (i_ref[k, b], 0, 0)),
            pl.BlockSpec((ne, hid), lambda b, k, i_ref, g_ref: (0, 0)),
            pl.BlockSpec((1, hid, dim),
                         lambda b, k, i_ref, g_ref: (i_ref[k, b], 0, 0)),
            pl.BlockSpec((ne, dim), lambda b, k, i_ref, g_ref: (0, 0)),
        ],
        out_specs=pl.BlockSpec((1, hw, dim), lambda b, k, i_ref, g_ref: (b, 0, 0)),
    )
    outp = pl.pallas_call(
        _expert_body,
        grid_spec=grid_spec,
        out_shape=jax.ShapeDtypeStruct((bs, hw, dim), jnp.float32),
    )(idx, gates, xt, W1, b1, W2, b2)

    return jnp.transpose(outp, (0, 2, 1)).reshape(bs, dim, h, w)
